# Initial kernel scaffold; baseline (speedup 1.0000x reference)
import jax
import jax.numpy as jnp
from jax.experimental import pallas as pl


def kernel(h, e, edge_index, atom_tables, bond_tables, W, gamma, beta, rW1, rb1, rW2, rb2, rW3, rb3):
    raise NotImplementedError("write your pallas kernel here")


# speedup vs baseline: 2.5335x; 2.5335x over previous
import functools

import jax
import jax.numpy as jnp
from jax import lax
from jax.experimental import pallas as pl
from jax.experimental.pallas import tpu as pltpu
from jax.experimental.pallas import tpu_sc as plsc

N = 10000
E = 320000
D = 128
NUM_ATOM_FEATS = 9
ATOM_VOCAB = 100
NUM_LAYERS = 4

NC = 2
NS = 16
NWORK = NC * NS
CH = 128
EPW = 10240
NCHUNK = EPW // CH
E_PAD = EPW * NWORK
N_PAD = 10112
ZROWS = N_PAD // NS

BN = 1000
GRID = N // BN


def _segsum_body(y_hbm, sidx_hbm, didx_hbm, zeros_hbm, out_hbm,
                 sidx_v, didx_v, gbuf, acc, sem):
    c = lax.axis_index("c")
    s = lax.axis_index("s")
    w = c * NS + s

    pltpu.sync_copy(zeros_hbm, acc.at[pl.ds(s * ZROWS, ZROWS)])
    pltpu.sync_copy(sidx_hbm.at[w], sidx_v)
    pltpu.sync_copy(didx_hbm.at[w], didx_v)
    plsc.subcore_barrier()

    def step(j, carry):
        pltpu.async_copy(y_hbm.at[sidx_v.at[j]], gbuf, sem).wait()
        pltpu.sync_copy(gbuf, acc.at[didx_v.at[j]], add=True)
        return carry

    lax.fori_loop(0, NCHUNK, step, 0)
    plsc.subcore_barrier()
    pltpu.sync_copy(acc.at[pl.ds(s * ZROWS, ZROWS)],
                    out_hbm.at[c, pl.ds(s * ZROWS, ZROWS)])


_segsum = functools.partial(
    pl.kernel,
    out_type=jax.ShapeDtypeStruct((NC, N_PAD, D), jnp.float32),
    mesh=plsc.VectorSubcoreMesh(core_axis_name="c", subcore_axis_name="s",
                                num_cores=NC, num_subcores=NS),
    scratch_types=[
        pltpu.VMEM((NCHUNK, CH), jnp.int32),
        pltpu.VMEM((NCHUNK, CH), jnp.int32),
        pltpu.VMEM((CH, D), jnp.float32),
        pltpu.VMEM_SHARED((N_PAD, D), jnp.float32),
        pltpu.SemaphoreType.DMA,
    ],
)(_segsum_body)


def _encoder_body(h_ref, tab_ref, degp_ref, x0_ref, y0_ref, dmat_ref):
    acc = jnp.zeros((BN, D), jnp.float32)
    iota = lax.broadcasted_iota(jnp.int32, (BN, ATOM_VOCAB), 1)
    for f in range(NUM_ATOM_FEATS):
        col = h_ref[:, f].reshape(BN, 1)
        oh = (col == iota).astype(jnp.float32)
        acc = acc + jnp.dot(oh, tab_ref[f], preferred_element_type=jnp.float32)
    deg = degp_ref[0] + degp_ref[1]
    dmat = lax.rsqrt(jnp.maximum(deg, 1.0))
    x0_ref[...] = acc
    dmat_ref[...] = dmat
    y0_ref[...] = acc * dmat


def _encoder(hm, tables, degp):
    return pl.pallas_call(
        _encoder_body,
        grid=(GRID,),
        in_specs=[
            pl.BlockSpec((BN, NUM_ATOM_FEATS), lambda i: (i, 0)),
            pl.BlockSpec((NUM_ATOM_FEATS, ATOM_VOCAB, D), lambda i: (0, 0, 0)),
            pl.BlockSpec((NC, BN, D), lambda i: (0, i, 0)),
        ],
        out_specs=[
            pl.BlockSpec((BN, D), lambda i: (i, 0)),
            pl.BlockSpec((BN, D), lambda i: (i, 0)),
            pl.BlockSpec((BN, D), lambda i: (i, 0)),
        ],
        out_shape=[jax.ShapeDtypeStruct((N, D), jnp.float32)] * 3,
    )(hm, tables, degp)


def _stage_a_body(s1p_ref, dmat_ref, x1_ref, y1_ref):
    dmat = dmat_ref[...]
    x1 = -((s1p_ref[0] + s1p_ref[1]) * dmat)
    x1_ref[...] = x1
    y1_ref[...] = x1 * dmat


def _stage_a(s1p, dmat):
    return pl.pallas_call(
        _stage_a_body,
        grid=(GRID,),
        in_specs=[
            pl.BlockSpec((NC, BN, D), lambda i: (0, i, 0)),
            pl.BlockSpec((BN, D), lambda i: (i, 0)),
        ],
        out_specs=[
            pl.BlockSpec((BN, D), lambda i: (i, 0)),
            pl.BlockSpec((BN, D), lambda i: (i, 0)),
        ],
        out_shape=[jax.ShapeDtypeStruct((N, D), jnp.float32)] * 2,
    )(s1p, dmat)


def _stage_b1_body(s2p_ref, dmat_ref, x0_ref, x1_ref, w_ref, hh_ref, stats_ref):
    i = pl.program_id(0)
    x0 = x0_ref[...]
    x2 = -2.0 * ((s2p_ref[0] + s2p_ref[1]) * dmat_ref[...]) - x0
    hh = (jnp.dot(x0, w_ref[0], preferred_element_type=jnp.float32)
          + jnp.dot(x1_ref[...], w_ref[1], preferred_element_type=jnp.float32)
          + jnp.dot(x2, w_ref[2], preferred_element_type=jnp.float32))
    hh_ref[...] = hh
    ssum = jnp.sum(hh, axis=0, keepdims=True)
    ssq = jnp.sum(hh * hh, axis=0, keepdims=True)
    upd = jnp.concatenate(
        [ssum, ssq, jnp.zeros((6, D), jnp.float32)], axis=0)

    @pl.when(i == 0)
    def _():
        stats_ref[...] = upd

    @pl.when(i > 0)
    def _():
        stats_ref[...] = stats_ref[...] + upd


def _stage_b1(s2p, dmat, x0, x1, wl):
    return pl.pallas_call(
        _stage_b1_body,
        grid=(GRID,),
        in_specs=[
            pl.BlockSpec((NC, BN, D), lambda i: (0, i, 0)),
            pl.BlockSpec((BN, D), lambda i: (i, 0)),
            pl.BlockSpec((BN, D), lambda i: (i, 0)),
            pl.BlockSpec((BN, D), lambda i: (i, 0)),
            pl.BlockSpec((3, D, D), lambda i: (0, 0, 0)),
        ],
        out_specs=[
            pl.BlockSpec((BN, D), lambda i: (i, 0)),
            pl.BlockSpec((8, D), lambda i: (0, 0)),
        ],
        out_shape=[
            jax.ShapeDtypeStruct((N, D), jnp.float32),
            jax.ShapeDtypeStruct((8, D), jnp.float32),
        ],
    )(s2p, dmat, x0, x1, wl)


def _stage_b2_body(hh_ref, stats_ref, x0_ref, gb_ref, dmat_ref,
                   x_ref, y_ref, xsum_ref):
    i = pl.program_id(0)
    mu = stats_ref[0:1, :] * (1.0 / N)
    var = stats_ref[1:2, :] * (1.0 / N) - mu * mu
    rstd = lax.rsqrt(var + 1e-5)
    hn = (hh_ref[...] - mu) * rstd * gb_ref[0:1, :] + gb_ref[1:2, :]
    x = x0_ref[...] + jnp.maximum(hn, 0.0)
    x_ref[...] = x
    y_ref[...] = x * dmat_ref[...]
    upd = jnp.concatenate(
        [jnp.sum(x, axis=0, keepdims=True), jnp.zeros((7, D), jnp.float32)],
        axis=0)

    @pl.when(i == 0)
    def _():
        xsum_ref[...] = upd

    @pl.when(i > 0)
    def _():
        xsum_ref[...] = xsum_ref[...] + upd


def _stage_b2(hh, stats, x0, gb, dmat):
    return pl.pallas_call(
        _stage_b2_body,
        grid=(GRID,),
        in_specs=[
            pl.BlockSpec((BN, D), lambda i: (i, 0)),
            pl.BlockSpec((8, D), lambda i: (0, 0)),
            pl.BlockSpec((BN, D), lambda i: (i, 0)),
            pl.BlockSpec((8, D), lambda i: (0, 0)),
            pl.BlockSpec((BN, D), lambda i: (i, 0)),
        ],
        out_specs=[
            pl.BlockSpec((BN, D), lambda i: (i, 0)),
            pl.BlockSpec((BN, D), lambda i: (i, 0)),
            pl.BlockSpec((8, D), lambda i: (0, 0)),
        ],
        out_shape=[
            jax.ShapeDtypeStruct((N, D), jnp.float32),
            jax.ShapeDtypeStruct((N, D), jnp.float32),
            jax.ShapeDtypeStruct((8, D), jnp.float32),
        ],
    )(hh, stats, x0, gb, dmat)


def _readout_body(xsum_ref, w1_ref, b1_ref, w2_ref, b2_ref, w3_ref, b3_ref,
                  y_ref):
    hg = xsum_ref[0:1, :] * (1.0 / N)
    y1 = jnp.maximum(
        jnp.dot(hg, w1_ref[...], preferred_element_type=jnp.float32)
        + b1_ref[...], 0.0)
    y2 = jnp.maximum(
        jnp.dot(y1, w2_ref[...], preferred_element_type=jnp.float32)
        + b2_ref[...], 0.0)
    y_ref[...] = (jnp.dot(y2, w3_ref[...], preferred_element_type=jnp.float32)
                  + b3_ref[...])


def _readout(xsum, w1, b1, w2, b2, w3, b3):
    return pl.pallas_call(
        _readout_body,
        out_shape=jax.ShapeDtypeStruct((1, b3.shape[-1]), jnp.float32),
    )(xsum, w1, b1, w2, b2, w3, b3)


def kernel(h, e, edge_index, atom_tables, bond_tables, W, gamma, beta,
           rW1, rb1, rW2, rb2, rW3, rb3):
    del e, bond_tables
    src = edge_index[0].astype(jnp.int32)
    dst = edge_index[1].astype(jnp.int32)
    pad = E_PAD - E
    srcp = jnp.concatenate([src, jnp.zeros((pad,), jnp.int32)])
    srcp = srcp.reshape(NWORK, NCHUNK, CH)
    dstp = jnp.concatenate([dst, jnp.full((pad,), N, jnp.int32)])
    dstp = dstp.reshape(NWORK, NCHUNK, CH)
    zrows = jnp.zeros((ZROWS, D), jnp.float32)

    degp = _segsum(jnp.ones((N, D), jnp.float32), srcp, dstp, zrows)
    x0, y, dmat = _encoder(h.astype(jnp.int32), atom_tables, degp)

    xsum = None
    for l in range(NUM_LAYERS):
        s1p = _segsum(y, srcp, dstp, zrows)
        x1, y1 = _stage_a(s1p, dmat)
        s2p = _segsum(y1, srcp, dstp, zrows)
        hh, stats = _stage_b1(s2p, dmat, x0, x1, W[l].reshape(3, D, D))
        gb = jnp.concatenate(
            [gamma[l].reshape(1, D), beta[l].reshape(1, D),
             jnp.zeros((6, D), jnp.float32)], axis=0)
        x0, y, xsum = _stage_b2(hh, stats, x0, gb, dmat)

    return _readout(xsum, rW1, rb1.reshape(1, -1), rW2, rb2.reshape(1, -1),
                    rW3, rb3.reshape(1, -1))
